# baseline (speedup 1.0000x reference)
import functools

import jax
import jax.numpy as jnp
from jax import lax
from jax.experimental import pallas as pl
from jax.experimental.pallas import tpu as pltpu
from jax.experimental.pallas import tpu_sc as plsc

N, D, H, C, K = 32768, 2048, 256, 16, 64
TOPK = 10
ROWS_PER_BLOCK = 1024
NUM_BLOCKS = N // ROWS_PER_BLOCK

L = 16
NS = 16
CHUNK = N // NS
NEG_INF = jnp.float32(-jnp.inf)
BIG = jnp.int32(2**30)


def _score_body(xa_ref, xb_ref, w_ref, out_ref):
    wa = w_ref[...][None, : D // 2]
    wb = w_ref[...][None, D // 2:]
    out_ref[...] = (jnp.sum(xa_ref[...] * wa, axis=1)
                    + jnp.sum(xb_ref[...] * wb, axis=1))


def _scores(x_path, w):
    return pl.pallas_call(
        _score_body,
        grid=(NUM_BLOCKS,),
        in_specs=[
            pl.BlockSpec((ROWS_PER_BLOCK, D // 2), lambda i: (i, 0)),
            pl.BlockSpec((ROWS_PER_BLOCK, D // 2), lambda i: (i, 1)),
            pl.BlockSpec((D,), lambda i: (0,)),
        ],
        out_specs=pl.BlockSpec((ROWS_PER_BLOCK,), lambda i: (i,)),
        out_shape=jax.ShapeDtypeStruct((N,), jnp.float32),
    )(x_path, x_path, w)


def _sc_topk_gather(scores, x_path):
    mesh = plsc.VectorSubcoreMesh(core_axis_name="c", subcore_axis_name="s")

    def _allmax(v, lane):
        for s in (1, 2, 4, 8):
            v = jnp.maximum(v, v[jnp.bitwise_xor(lane, s)])
        return v

    def _allmin(v, lane):
        for s in (1, 2, 4, 8):
            v = jnp.minimum(v, v[jnp.bitwise_xor(lane, s)])
        return v

    @functools.partial(
        pl.kernel,
        mesh=mesh,
        compiler_params=pltpu.CompilerParams(needs_layout_passes=False),
        out_type=jax.ShapeDtypeStruct((L, D), jnp.float32),
        scratch_types=[
            pltpu.VMEM((CHUNK,), jnp.float32),
            pltpu.VMEM((L,), jnp.float32),
            pltpu.VMEM((L,), jnp.int32),
            pltpu.VMEM_SHARED((NS * L,), jnp.float32),
            pltpu.VMEM_SHARED((NS * L,), jnp.int32),
            pltpu.VMEM((NS * L,), jnp.float32),
            pltpu.VMEM((NS * L,), jnp.int32),
            pltpu.VMEM((L,), jnp.int32),
            pltpu.VMEM((L, D), jnp.float32),
            pltpu.SemaphoreType.DMA,
        ],
    )
    def body(scores_hbm, x_hbm, rows_hbm, s_v, sel_s_v, sel_i_v,
             sh_s, sh_i, cand_s_v, cand_i_v, idx_v, rows_v, sem):
        cid = lax.axis_index("c")
        sid = lax.axis_index("s")
        lane = lax.iota(jnp.int32, 16)

        @pl.when(cid == 0)
        def _():
            base = sid * CHUNK
            pltpu.sync_copy(scores_hbm.at[pl.ds(base, CHUNK)], s_v)

            acc = jnp.full((L,), NEG_INF, jnp.float32)
            aidx = jnp.full((L,), BIG, jnp.int32)
            for j in range(CHUNK // L):
                v = s_v[pl.ds(j * L, L)]
                vidx = (base + j * L) + lane
                take = v > acc
                acc = jnp.where(take, v, acc)
                aidx = jnp.where(take, vidx, aidx)

            sel_s = jnp.full((L,), NEG_INF, jnp.float32)
            sel_i = jnp.full((L,), BIG, jnp.int32)
            for t in range(TOPK):
                m = _allmax(acc, lane)
                g = _allmin(jnp.where(acc == m, aidx, BIG), lane)
                sel_s = jnp.where(lane == t, m, sel_s)
                sel_i = jnp.where(lane == t, g, sel_i)
                p = g - base
                l0 = jnp.bitwise_and(p, L - 1)
                plsc.store_scatter(
                    s_v, [p], jnp.full((L,), NEG_INF, jnp.float32),
                    mask=lane == 0)
                cmax = jnp.full((L,), NEG_INF, jnp.float32)
                cidx = jnp.full((L,), BIG, jnp.int32)
                for blk in range(CHUNK // (L * L)):
                    pos = l0 + L * (blk * L + lane)
                    cv = plsc.load_gather(s_v, [pos])
                    cvi = base + pos
                    take = (cv > cmax) | ((cv == cmax) & (cvi < cidx))
                    cmax = jnp.where(take, cv, cmax)
                    cidx = jnp.where(take, cvi, cidx)
                cm = _allmax(cmax, lane)
                cg = _allmin(jnp.where(cmax == cm, cidx, BIG), lane)
                lm = lane == l0
                acc = jnp.where(lm, cm, acc)
                aidx = jnp.where(lm, cg, aidx)

            sel_s_v[...] = sel_s
            sel_i_v[...] = sel_i
            pltpu.sync_copy(sel_s_v, sh_s.at[pl.ds(sid * L, L)])
            pltpu.sync_copy(sel_i_v, sh_i.at[pl.ds(sid * L, L)])
            plsc.subcore_barrier()

            @pl.when(sid == 0)
            def _():
                pltpu.sync_copy(sh_s, cand_s_v)
                pltpu.sync_copy(sh_i, cand_i_v)
                acc2 = jnp.full((L,), NEG_INF, jnp.float32)
                aidx2 = jnp.full((L,), BIG, jnp.int32)
                for r in range(NS):
                    v = cand_s_v[pl.ds(r * L, L)]
                    vi = cand_i_v[pl.ds(r * L, L)]
                    take = (v > acc2) | ((v == acc2) & (vi < aidx2))
                    acc2 = jnp.where(take, v, acc2)
                    aidx2 = jnp.where(take, vi, aidx2)
                chosen = jnp.full((L,), 0, jnp.int32)
                for t in range(TOPK):
                    m = _allmax(acc2, lane)
                    g = _allmin(jnp.where(acc2 == m, aidx2, BIG), lane)
                    chosen = jnp.where(lane == t, g, chosen)
                    w = lax.shift_right_logical(g, 11)
                    pos0 = w * L + lane
                    slot_i = plsc.load_gather(cand_i_v, [pos0])
                    hit = slot_i == g
                    plsc.store_scatter(
                        cand_s_v, [pos0],
                        jnp.full((L,), NEG_INF, jnp.float32), mask=hit)
                    l0 = _allmin(jnp.where(hit, lane, BIG), lane)
                    cmax = jnp.full((L,), NEG_INF, jnp.float32)
                    cidx = jnp.full((L,), BIG, jnp.int32)
                    pos = l0 + L * lane
                    cv = plsc.load_gather(cand_s_v, [pos])
                    cvi = plsc.load_gather(cand_i_v, [pos])
                    take = (cv > cmax) | ((cv == cmax) & (cvi < cidx))
                    cmax = jnp.where(take, cv, cmax)
                    cidx = jnp.where(take, cvi, cidx)
                    cm = _allmax(cmax, lane)
                    cg = _allmin(jnp.where(cmax == cm, cidx, BIG), lane)
                    lm = lane == l0
                    acc2 = jnp.where(lm, cm, acc2)
                    aidx2 = jnp.where(lm, cg, aidx2)

                idx_v[...] = chosen
                pltpu.async_copy(x_hbm.at[idx_v], rows_v, sem).wait()
                pltpu.sync_copy(rows_v, rows_hbm)

    return body(scores, x_path)


def _tail_body(rows_ref, proto_ref, w2_ref, b2_ref, wr_ref, br_ref,
               wc_ref, bc_ref, bag_ref, prob_ref, yhat_ref, sim_ref):
    mrows = rows_ref[...]
    dn = (((1,), (1,)), ((), ()))
    f = lax.dot_general(mrows, w2_ref[...], dn,
                        preferred_element_type=jnp.float32) + b2_ref[...][None, :]
    p = lax.dot_general(proto_ref[...], w2_ref[...], dn,
                        preferred_element_type=jnp.float32) + b2_ref[...][None, :]

    sim_rows = []
    for t in range(TOPK):
        d = f[t:t + 1, :] - p + 1e-6
        sim_rows.append(jnp.sqrt(jnp.sum(d * d, axis=1))[None, :])
    sim = jnp.concatenate(sim_rows, axis=0)
    cmax = jnp.max(sim, axis=1, keepdims=True)
    sim = sim / cmax
    sim_coding = jnp.mean(sim, axis=0, keepdims=True)

    h = lax.dot_general(sim_coding, wr_ref[...], dn,
                        preferred_element_type=jnp.float32) + br_ref[...][None, :]
    h = jnp.maximum(h, 0.0)
    bag = lax.dot_general(h, wc_ref[...], dn,
                          preferred_element_type=jnp.float32) + bc_ref[...][None, :]
    prob = jax.nn.softmax(bag, axis=1)

    bag_ref[...] = bag
    prob_ref[...] = prob
    yhat_ref[...] = jnp.where(prob[:, 1:2] > prob[:, 0:1], 1, 0).astype(jnp.int32)
    sim_ref[...] = sim_coding


def _tail(rows, prototype, W2, b2, Wr, br, Wc, bc):
    out_shapes = (
        jax.ShapeDtypeStruct((1, 2), jnp.float32),
        jax.ShapeDtypeStruct((1, 2), jnp.float32),
        jax.ShapeDtypeStruct((1, 1), jnp.int32),
        jax.ShapeDtypeStruct((1, K), jnp.float32),
    )
    vmem = lambda: pl.BlockSpec(memory_space=pltpu.MemorySpace.VMEM)
    return pl.pallas_call(
        _tail_body,
        in_specs=[vmem()] * 8,
        out_specs=(vmem(), vmem(), vmem(), vmem()),
        out_shape=out_shapes,
    )(rows, prototype, W2, b2, Wr, br, Wc, bc)


def kernel(x_path, prototype, W3, b3, W2, b2, Wr, br, Wc, bc):
    w = W3[1] - W3[0]
    scores = _scores(x_path, w)
    rows = _sc_topk_gather(scores, x_path)
    bag, prob, yhat, sim_coding = _tail(
        rows, prototype, W2, b2, Wr, br, Wc, bc)
    return (bag, prob, yhat.reshape(1), sim_coding)


# speedup vs baseline: 1.2795x; 1.2795x over previous
import jax
import jax.numpy as jnp
from jax import lax
from jax.experimental import pallas as pl
from jax.experimental.pallas import tpu as pltpu

N, D, H, C, K = 32768, 2048, 256, 16, 64
TOPK = 10
ROWS_PER_BLOCK = 1024
NUM_BLOCKS = N // ROWS_PER_BLOCK


def _body(x_ref, w_ref, x_hbm, proto_ref, w2_ref, b2_ref, wr_ref, br_ref,
          wc_ref, bc_ref, bag_ref, prob_ref, yhat_ref, sim_ref,
          scores_v, m_scratch, sem):
    i = pl.program_id(0)

    @pl.when(i < NUM_BLOCKS)
    def _score():
        part = jnp.sum(x_ref[...] * w_ref[...][None, :], axis=1)
        scores_v[pl.ds(i * ROWS_PER_BLOCK, ROWS_PER_BLOCK)] = part

    @pl.when(i == NUM_BLOCKS)
    def _tail():
        s = scores_v[...].reshape(N // 128, 128)
        rows = lax.broadcasted_iota(jnp.int32, (N // 128, 128), 0)
        cols = lax.broadcasted_iota(jnp.int32, (N // 128, 128), 1)
        lin = rows * 128 + cols

        copies = []
        for t in range(TOPK):
            m = jnp.max(s)
            idx = jnp.min(jnp.where(s == m, lin, jnp.int32(N)))
            cp = pltpu.make_async_copy(
                x_hbm.at[pl.ds(idx, 1), :], m_scratch.at[pl.ds(t, 1), :], sem)
            cp.start()
            copies.append(cp)
            s = jnp.where(lin == idx, -jnp.inf, s)

        dn = (((1,), (1,)), ((), ()))
        p = lax.dot_general(proto_ref[...], w2_ref[...], dn,
                            preferred_element_type=jnp.float32) + b2_ref[...][None, :]
        for cp in copies:
            cp.wait()

        mrows = m_scratch[...]
        f = lax.dot_general(mrows, w2_ref[...], dn,
                            preferred_element_type=jnp.float32) + b2_ref[...][None, :]

        diff = f[:, None, :] - p[None, :, :] + 1e-6
        sim = jnp.sqrt(jnp.sum(diff * diff, axis=2))
        cmax = jnp.max(sim, axis=1, keepdims=True)
        sim = sim / cmax
        sim_coding = jnp.mean(sim, axis=0, keepdims=True)

        h = lax.dot_general(sim_coding, wr_ref[...], dn,
                            preferred_element_type=jnp.float32) + br_ref[...][None, :]
        h = jnp.maximum(h, 0.0)
        bag = lax.dot_general(h, wc_ref[...], dn,
                              preferred_element_type=jnp.float32) + bc_ref[...][None, :]
        prob = jax.nn.softmax(bag, axis=1)

        bag_ref[...] = bag
        prob_ref[...] = prob
        yhat_ref[...] = jnp.where(prob[:, 1:2] > prob[:, 0:1], 1, 0).astype(jnp.int32)
        sim_ref[...] = sim_coding


def kernel(x_path, prototype, W3, b3, W2, b2, Wr, br, Wc, bc):
    w = W3[1] - W3[0]
    out_shapes = (
        jax.ShapeDtypeStruct((1, 2), jnp.float32),
        jax.ShapeDtypeStruct((1, 2), jnp.float32),
        jax.ShapeDtypeStruct((1, 1), jnp.int32),
        jax.ShapeDtypeStruct((1, K), jnp.float32),
    )
    vmem = lambda: pl.BlockSpec(memory_space=pltpu.MemorySpace.VMEM)
    last = NUM_BLOCKS - 1
    bag, prob, yhat, sim_coding = pl.pallas_call(
        _body,
        grid=(NUM_BLOCKS + 1,),
        in_specs=[
            pl.BlockSpec((ROWS_PER_BLOCK, D),
                         lambda i: (jnp.minimum(i, last), 0)),
            pl.BlockSpec((D,), lambda i: (0,)),
            pl.BlockSpec(memory_space=pltpu.MemorySpace.HBM),
            vmem(), vmem(), vmem(), vmem(), vmem(), vmem(), vmem(),
        ],
        out_specs=(
            pl.BlockSpec((1, 2), lambda i: (0, 0)),
            pl.BlockSpec((1, 2), lambda i: (0, 0)),
            pl.BlockSpec((1, 1), lambda i: (0, 0)),
            pl.BlockSpec((1, K), lambda i: (0, 0)),
        ),
        out_shape=out_shapes,
        scratch_shapes=[
            pltpu.VMEM((N,), jnp.float32),
            pltpu.VMEM((TOPK, D), jnp.float32),
            pltpu.SemaphoreType.DMA,
        ],
    )(x_path, w, x_path, prototype, W2, b2, Wr, br, Wc, bc)
    return (bag, prob, yhat.reshape(1), sim_coding)
